# Initial kernel scaffold; baseline (speedup 1.0000x reference)
import jax
import jax.numpy as jnp
from jax.experimental import pallas as pl


def kernel(x, edge_index, Wl1, bl1, Wr1, Wl2, bl2, Wr2, Wl3, bl3, Wr3):
    raise NotImplementedError("write your pallas kernel here")


# speedup vs baseline: 4.6737x; 4.6737x over previous
import functools

import jax
import jax.numpy as jnp
from jax import lax
from jax.experimental import pallas as pl
from jax.experimental.pallas import tpu as pltpu
from jax.experimental.pallas import tpu_sc as plsc

N = 10000
E = 320000
NP = 10240
NC = 2
NS = 16
NW = NC * NS
ET = E // NW
K = 80
NCHUNK = ET // K
RPT = NP // NS


@functools.lru_cache(maxsize=None)
def _make_sc_agg(D: int, n_pre: int, with_counts: bool):
  mesh = plsc.VectorSubcoreMesh(
      core_axis_name="c", subcore_axis_name="s", num_cores=NC,
      num_subcores=NS)

  out_type = [jax.ShapeDtypeStruct((NC, NP, D), jnp.float32)] * n_pre
  if with_counts:
    out_type.append(jax.ShapeDtypeStruct((NC, NP, 16), jnp.float32))

  scratch_types = [
      pltpu.VMEM_SHARED((NP, D), jnp.float32),
      pltpu.VMEM((K,), jnp.int32),
      pltpu.VMEM((K,), jnp.int32),
      pltpu.VMEM((K, D), jnp.float32),
      pltpu.VMEM((RPT, D), jnp.float32),
      pltpu.SemaphoreType.DMA,
  ]
  if with_counts:
    scratch_types += [
        pltpu.VMEM((K, 16), jnp.float32),
        pltpu.VMEM((RPT, 16), jnp.float32),
        pltpu.VMEM_SHARED((NP, 16), jnp.float32),
    ]

  def body(*refs):
    pres = refs[:n_pre]
    src, dst, zeros = refs[n_pre:n_pre + 3]
    k = n_pre + 3
    if with_counts:
      zeros16, ones = refs[k:k + 2]
      k += 2
    outs = refs[k:k + n_pre]
    k += n_pre
    if with_counts:
      out_cnt = refs[k]
      k += 1
    acc, idx_s, idx_d, rows, zbuf, sem = refs[k:k + 6]
    k += 6
    if with_counts:
      ones_v, zbuf16, cnt = refs[k:k + 3]

    c = lax.axis_index("c")
    s = lax.axis_index("s")
    wid = s * NC + c
    rbase = s * RPT
    ebase = wid * ET

    for phase in range(n_pre):
      pre = pres[phase]
      do_cnt = with_counts and phase == 0

      pltpu.sync_copy(zeros, zbuf)
      pltpu.sync_copy(zbuf, acc.at[pl.ds(rbase, RPT)])
      if do_cnt:
        pltpu.sync_copy(zeros16, zbuf16)
        pltpu.sync_copy(zbuf16, cnt.at[pl.ds(rbase, RPT)])
        pltpu.sync_copy(ones, ones_v)
      plsc.subcore_barrier()

      def chunk(g, carry):
        off = ebase + g * K
        pltpu.sync_copy(src.at[pl.ds(off, K)], idx_s)
        pltpu.sync_copy(dst.at[pl.ds(off, K)], idx_d)
        pltpu.async_copy(pre.at[idx_s], rows, sem).wait()
        pltpu.sync_copy(rows, acc.at[idx_d], add=True)
        if do_cnt:
          pltpu.sync_copy(ones_v, cnt.at[idx_d], add=True)
        return carry

      lax.fori_loop(0, NCHUNK, chunk, 0)
      plsc.subcore_barrier()

      pltpu.sync_copy(acc.at[pl.ds(rbase, RPT)], zbuf)
      pltpu.sync_copy(zbuf, outs[phase].at[c, pl.ds(rbase, RPT)])
      if do_cnt:
        pltpu.sync_copy(cnt.at[pl.ds(rbase, RPT)], zbuf16)
        pltpu.sync_copy(zbuf16, out_cnt.at[c, pl.ds(rbase, RPT)])
      if phase + 1 < n_pre:
        plsc.subcore_barrier()

  return pl.kernel(body, out_type=tuple(out_type), mesh=mesh,
                   scratch_types=scratch_types,
                   compiler_params=pltpu.CompilerParams(
                       use_tc_tiling_on_sc=False))


_BM = 2560


def _tc1_body(x_ref, wl_ref, wr_ref, b_ref, prea_ref, preb_ref, r_ref):
  xb = x_ref[...]
  pre = jnp.dot(xb, wl_ref[...], preferred_element_type=jnp.float32)
  prea_ref[...] = pre[:, :64]
  preb_ref[...] = pre[:, 64:]
  r_ref[...] = (jnp.dot(xb, wr_ref[...], preferred_element_type=jnp.float32)
                + b_ref[...])


def _make_tc_mid_body(n_parts):
  def tc_mid_body(*refs):
    part_refs = refs[:n_parts]
    pcnt_ref, r_ref, wl_ref, wr_ref, b_ref, pre_ref, rn_ref = refs[n_parts:]
    cnt = jnp.maximum(pcnt_ref[0, :, 0:1] + pcnt_ref[1, :, 0:1], 1.0)
    mean = jnp.concatenate([p[0] + p[1] for p in part_refs], axis=1) / cnt
    h = jnp.maximum(mean + r_ref[...], 0.0)
    pre_ref[...] = jnp.dot(h, wl_ref[...], preferred_element_type=jnp.float32)
    rn_ref[...] = (jnp.dot(h, wr_ref[...],
                           preferred_element_type=jnp.float32) + b_ref[...])
  return tc_mid_body


def _tc4_body(part_ref, pcnt_ref, r_ref, out_ref):
  p = part_ref[0, :, 0:1] + part_ref[1, :, 0:1]
  cnt = jnp.maximum(pcnt_ref[0, :, 0:1] + pcnt_ref[1, :, 0:1], 1.0)
  out_ref[...] = p / cnt + r_ref[...]


def _row_spec(d):
  return pl.BlockSpec((_BM, d), lambda i: (i, 0))


def _part_spec(d):
  return pl.BlockSpec((NC, _BM, d), lambda i: (0, i, 0))


def _full_spec(a, b):
  return pl.BlockSpec((a, b), lambda i: (0, 0))


def _tc1(x, wl, wr, b):
  return pl.pallas_call(
      _tc1_body,
      grid=(NP // _BM,),
      in_specs=[_row_spec(128), _full_spec(128, 128),
                _full_spec(128, 128), _full_spec(1, 128)],
      out_specs=[_row_spec(64), _row_spec(64), _row_spec(128)],
      out_shape=[jax.ShapeDtypeStruct((N, 64), jnp.float32),
                 jax.ShapeDtypeStruct((N, 64), jnp.float32),
                 jax.ShapeDtypeStruct((N, 128), jnp.float32)],
  )(x, wl, wr, b)


def _tc_mid(parts, pcnt, r, wl, wr, b):
  d_in, d_out_l = wl.shape
  d_out_r = wr.shape[1]
  dp = d_in // len(parts)
  return pl.pallas_call(
      _make_tc_mid_body(len(parts)),
      grid=(NP // _BM,),
      in_specs=[_part_spec(dp)] * len(parts)
      + [_part_spec(16), _row_spec(d_in), _full_spec(d_in, d_out_l),
         _full_spec(d_in, d_out_r), _full_spec(1, d_out_r)],
      out_specs=[_row_spec(d_out_l), _row_spec(d_out_r)],
      out_shape=[jax.ShapeDtypeStruct((N, d_out_l), jnp.float32),
                 jax.ShapeDtypeStruct((N, d_out_r), jnp.float32)],
  )(*parts, pcnt, r, wl, wr, b)


def _tc4(part, pcnt, r):
  return pl.pallas_call(
      _tc4_body,
      grid=(NP // _BM,),
      in_specs=[_part_spec(16), _part_spec(16), _row_spec(1)],
      out_specs=_row_spec(1),
      out_shape=jax.ShapeDtypeStruct((N, 1), jnp.float32),
  )(part, pcnt, r)


def kernel(x, edge_index, Wl1, bl1, Wr1, Wl2, bl2, Wr2, Wl3, bl3, Wr3):
  src = edge_index[0].astype(jnp.int32)
  dst = edge_index[1].astype(jnp.int32)

  z64 = jnp.zeros((RPT, 64), jnp.float32)
  z16 = jnp.zeros((RPT, 16), jnp.float32)
  ones = jnp.ones((K, 16), jnp.float32)

  pre1a, pre1b, r1 = _tc1(x, Wl1, Wr1, bl1.reshape(1, -1))
  part1a, part1b, pcnt = _make_sc_agg(64, 2, True)(
      pre1a, pre1b, src, dst, z64, z16, ones)

  pre2, r2 = _tc_mid([part1a, part1b], pcnt, r1, Wl2, Wr2, bl2.reshape(1, -1))
  (part2,) = _make_sc_agg(64, 1, False)(pre2, src, dst, z64)

  wl3p = jnp.concatenate([Wl3, jnp.zeros((Wl3.shape[0], 15), jnp.float32)],
                         axis=1)
  pr3, rr3 = _tc_mid([part2], pcnt, r2, wl3p, Wr3, bl3.reshape(1, 1))
  (part3,) = _make_sc_agg(16, 1, False)(pr3, src, dst, z16)

  return _tc4(part3, pcnt, rr3)


# speedup vs baseline: 5.9012x; 1.2627x over previous
import functools

import jax
import jax.numpy as jnp
from jax import lax
from jax.experimental import pallas as pl
from jax.experimental.pallas import tpu as pltpu
from jax.experimental.pallas import tpu_sc as plsc

N = 10000
E = 320000
NP = 10240
NC = 2
NS = 16
NW = NC * NS
K = 128
NCHUNK = 80
EP = NW * NCHUNK * K
ET = EP // NW
RPT = NP // NS
ZR = 128


@functools.lru_cache(maxsize=None)
def _make_sc_agg(D: int, n_pre: int, with_counts: bool):
  mesh = plsc.VectorSubcoreMesh(
      core_axis_name="c", subcore_axis_name="s", num_cores=NC,
      num_subcores=NS)

  out_type = [jax.ShapeDtypeStruct((NC, NP, D), jnp.float32)] * n_pre
  if with_counts:
    out_type.append(jax.ShapeDtypeStruct((NC, NP, 16), jnp.float32))

  scratch_types = [
      pltpu.VMEM_SHARED((NP, D), jnp.float32),
      pltpu.VMEM((NCHUNK, K), jnp.int32),
      pltpu.VMEM((NCHUNK, K), jnp.int32),
      pltpu.VMEM((K, D), jnp.float32),
      pltpu.VMEM((K, D), jnp.float32),
      pltpu.VMEM((ZR, D), jnp.float32),
      pltpu.SemaphoreType.DMA,
      pltpu.SemaphoreType.DMA,
  ]
  if with_counts:
    scratch_types += [
        pltpu.VMEM((K, 16), jnp.float32),
        pltpu.VMEM((ZR, 16), jnp.float32),
        pltpu.VMEM_SHARED((NP, 16), jnp.float32),
    ]

  def body(*refs):
    pres = refs[:n_pre]
    src, dst, zeros = refs[n_pre:n_pre + 3]
    k = n_pre + 3
    if with_counts:
      zeros16, ones = refs[k:k + 2]
      k += 2
    outs = refs[k:k + n_pre]
    k += n_pre
    if with_counts:
      out_cnt = refs[k]
      k += 1
    acc, idx_s, idx_d, rows_a, rows_b, zbuf, sem_a, sem_b = refs[k:k + 8]
    k += 8
    if with_counts:
      ones_v, zbuf16, cnt = refs[k:k + 3]

    c = lax.axis_index("c")
    s = lax.axis_index("s")
    wid = s * NC + c
    rbase = s * RPT

    pltpu.sync_copy(src.at[wid], idx_s)
    pltpu.sync_copy(dst.at[wid], idx_d)
    if with_counts:
      pltpu.sync_copy(ones, ones_v)

    for phase in range(n_pre):
      pre = pres[phase]
      do_cnt = with_counts and phase == 0

      pltpu.sync_copy(zeros, zbuf)
      if do_cnt:
        pltpu.sync_copy(zeros16, zbuf16)
      for j in range(RPT // ZR):
        pltpu.sync_copy(zbuf, acc.at[pl.ds(rbase + j * ZR, ZR)])
        if do_cnt:
          pltpu.sync_copy(zbuf16, cnt.at[pl.ds(rbase + j * ZR, ZR)])
      plsc.subcore_barrier()

      def gather(g, buf, sem):
        return pltpu.async_copy(pre.at[idx_s.at[g]], buf, sem)

      def gwait(buf, sem):
        pltpu.make_async_copy(pre.at[idx_s.at[0]], buf, sem).wait()

      def scat(g, buf):
        pltpu.sync_copy(buf, acc.at[idx_d.at[g]], add=True)
        if do_cnt:
          pltpu.sync_copy(ones_v, cnt.at[idx_d.at[g]], add=True)

      gather(0, rows_a, sem_a)

      def chunk2(i, carry):
        g = 2 * i
        gather(g + 1, rows_b, sem_b)
        gwait(rows_a, sem_a)
        scat(g, rows_a)
        gather(g + 2, rows_a, sem_a)
        gwait(rows_b, sem_b)
        scat(g + 1, rows_b)
        return carry

      lax.fori_loop(0, NCHUNK // 2 - 1, chunk2, 0)
      g_last = NCHUNK - 2
      gather(g_last + 1, rows_b, sem_b)
      gwait(rows_a, sem_a)
      scat(g_last, rows_a)
      gwait(rows_b, sem_b)
      scat(g_last + 1, rows_b)
      plsc.subcore_barrier()

      for j in range(RPT // ZR):
        pltpu.sync_copy(acc.at[pl.ds(rbase + j * ZR, ZR)], zbuf)
        pltpu.sync_copy(zbuf, outs[phase].at[c, pl.ds(rbase + j * ZR, ZR)])
        if do_cnt:
          pltpu.sync_copy(cnt.at[pl.ds(rbase + j * ZR, ZR)], zbuf16)
          pltpu.sync_copy(zbuf16,
                          out_cnt.at[c, pl.ds(rbase + j * ZR, ZR)])
      if phase + 1 < n_pre:
        plsc.subcore_barrier()

  return pl.kernel(body, out_type=tuple(out_type), mesh=mesh,
                   scratch_types=scratch_types,
                   compiler_params=pltpu.CompilerParams(
                       use_tc_tiling_on_sc=False))


_BM = 2560


def _tc1_body(x_ref, wl_ref, wr_ref, b_ref, prea_ref, preb_ref, r_ref):
  xb = x_ref[...]
  pre = jnp.dot(xb, wl_ref[...], preferred_element_type=jnp.float32)
  prea_ref[...] = pre[:, :64]
  preb_ref[...] = pre[:, 64:]
  r_ref[...] = (jnp.dot(xb, wr_ref[...], preferred_element_type=jnp.float32)
                + b_ref[...])


def _make_tc_mid_body(n_parts):
  def tc_mid_body(*refs):
    part_refs = refs[:n_parts]
    pcnt_ref, r_ref, wl_ref, wr_ref, b_ref, pre_ref, rn_ref = refs[n_parts:]
    cnt = jnp.maximum(pcnt_ref[0, :, 0:1] + pcnt_ref[1, :, 0:1], 1.0)
    mean = jnp.concatenate([p[0] + p[1] for p in part_refs], axis=1) / cnt
    h = jnp.maximum(mean + r_ref[...], 0.0)
    pre_ref[...] = jnp.dot(h, wl_ref[...], preferred_element_type=jnp.float32)
    rn_ref[...] = (jnp.dot(h, wr_ref[...],
                           preferred_element_type=jnp.float32) + b_ref[...])
  return tc_mid_body


def _tc4_body(part_ref, pcnt_ref, r_ref, out_ref):
  p = part_ref[0, :, 0:1] + part_ref[1, :, 0:1]
  cnt = jnp.maximum(pcnt_ref[0, :, 0:1] + pcnt_ref[1, :, 0:1], 1.0)
  out_ref[...] = p / cnt + r_ref[...]


def _row_spec(d):
  return pl.BlockSpec((_BM, d), lambda i: (i, 0))


def _part_spec(d):
  return pl.BlockSpec((NC, _BM, d), lambda i: (0, i, 0))


def _full_spec(a, b):
  return pl.BlockSpec((a, b), lambda i: (0, 0))


def _tc1(x, wl, wr, b):
  return pl.pallas_call(
      _tc1_body,
      grid=(NP // _BM,),
      in_specs=[_row_spec(128), _full_spec(128, 128),
                _full_spec(128, 128), _full_spec(1, 128)],
      out_specs=[_row_spec(64), _row_spec(64), _row_spec(128)],
      out_shape=[jax.ShapeDtypeStruct((N, 64), jnp.float32),
                 jax.ShapeDtypeStruct((N, 64), jnp.float32),
                 jax.ShapeDtypeStruct((N, 128), jnp.float32)],
  )(x, wl, wr, b)


def _tc_mid(parts, pcnt, r, wl, wr, b):
  d_in, d_out_l = wl.shape
  d_out_r = wr.shape[1]
  dp = d_in // len(parts)
  return pl.pallas_call(
      _make_tc_mid_body(len(parts)),
      grid=(NP // _BM,),
      in_specs=[_part_spec(dp)] * len(parts)
      + [_part_spec(16), _row_spec(d_in), _full_spec(d_in, d_out_l),
         _full_spec(d_in, d_out_r), _full_spec(1, d_out_r)],
      out_specs=[_row_spec(d_out_l), _row_spec(d_out_r)],
      out_shape=[jax.ShapeDtypeStruct((N, d_out_l), jnp.float32),
                 jax.ShapeDtypeStruct((N, d_out_r), jnp.float32)],
  )(*parts, pcnt, r, wl, wr, b)


def _tc4(part, pcnt, r):
  return pl.pallas_call(
      _tc4_body,
      grid=(NP // _BM,),
      in_specs=[_part_spec(16), _part_spec(16), _row_spec(1)],
      out_specs=_row_spec(1),
      out_shape=jax.ShapeDtypeStruct((N, 1), jnp.float32),
  )(part, pcnt, r)


def kernel(x, edge_index, Wl1, bl1, Wr1, Wl2, bl2, Wr2, Wl3, bl3, Wr3):
  pad = EP - E
  src = jnp.concatenate(
      [edge_index[0].astype(jnp.int32), jnp.zeros((pad,), jnp.int32)]
  ).reshape(NW, NCHUNK, K)
  dst = jnp.concatenate(
      [edge_index[1].astype(jnp.int32), jnp.full((pad,), N, jnp.int32)]
  ).reshape(NW, NCHUNK, K)

  z64 = jnp.zeros((ZR, 64), jnp.float32)
  z16 = jnp.zeros((ZR, 16), jnp.float32)
  ones = jnp.ones((K, 16), jnp.float32)

  pre1a, pre1b, r1 = _tc1(x, Wl1, Wr1, bl1.reshape(1, -1))
  part1a, part1b, pcnt = _make_sc_agg(64, 2, True)(
      pre1a, pre1b, src, dst, z64, z16, ones)

  pre2, r2 = _tc_mid([part1a, part1b], pcnt, r1, Wl2, Wr2, bl2.reshape(1, -1))
  (part2,) = _make_sc_agg(64, 1, False)(pre2, src, dst, z64)

  wl3p = jnp.concatenate([Wl3, jnp.zeros((Wl3.shape[0], 15), jnp.float32)],
                         axis=1)
  pr3, rr3 = _tc_mid([part2], pcnt, r2, wl3p, Wr3, bl3.reshape(1, 1))
  (part3,) = _make_sc_agg(16, 1, False)(pr3, src, dst, z16)

  return _tc4(part3, pcnt, rr3)
